# baseline (speedup 1.0000x reference)
import functools

import jax
import jax.numpy as jnp
from jax import lax
from jax.experimental import pallas as pl
from jax.experimental.pallas import tpu as pltpu
from jax.experimental.pallas import tpu_sc as plsc

_NC = 2
_NS = 16
_L = 16
_NW = _NC * _NS


@functools.lru_cache(maxsize=None)
def _make_sc_kernel(B, D, chunk_sizes):
    b_per_w = B // _NW
    assert sum(chunk_sizes) == b_per_w
    starts = []
    off = 0
    for sz in chunk_sizes:
        starts.append(off)
        off += sz
    cmax = max(chunk_sizes)
    n_chunks = len(chunk_sizes)
    mesh = plsc.VectorSubcoreMesh(core_axis_name="c", subcore_axis_name="s")

    @functools.partial(
        pl.kernel,
        mesh=mesh,
        out_type=jax.ShapeDtypeStruct((_NW, _L), jnp.float32),
        scratch_types=[
            pltpu.VMEM((b_per_w,), jnp.int32),
            pltpu.VMEM((b_per_w, D), jnp.float32),
            pltpu.VMEM((cmax, D), jnp.float32),
            pltpu.VMEM((cmax, D), jnp.float32),
            pltpu.VMEM((cmax, D), jnp.float32),
            pltpu.VMEM((_L,), jnp.float32),
            pltpu.SemaphoreType.DMA,
            pltpu.SemaphoreType.DMA,
            pltpu.SemaphoreType.DMA,
        ],
        compiler_params=pltpu.CompilerParams(needs_layout_passes=False),
    )
    def sc_kernel(x_hbm, lab_hbm, cen_hbm, out_hbm,
                  idx_v, x_v, c_v0, c_v1, c_v2, acc_v,
                  sem0, sem1, sem2):
        wid = lax.axis_index("s") * _NC + lax.axis_index("c")
        base = wid * b_per_w
        cbufs = ((c_v0, sem0), (c_v1, sem1), (c_v2, sem2))

        pltpu.sync_copy(lab_hbm.at[pl.ds(base, b_per_w)], idx_v)

        def start(ci):
            cb, sem = cbufs[ci % 3]
            r0, sz = starts[ci], chunk_sizes[ci]
            hc = pltpu.async_copy(
                cen_hbm.at[idx_v.at[pl.ds(r0, sz)]], cb.at[pl.ds(0, sz)], sem)
            hx = pltpu.async_copy(
                x_hbm.at[pl.ds(base + r0, sz)], x_v.at[pl.ds(r0, sz)], sem)
            return hc, hx

        handles = [start(0), start(1)]
        tot = jnp.float32(0.0)
        for ci in range(n_chunks):
            if ci + 2 < n_chunks:
                handles.append(start(ci + 2))
            pending = handles[ci]
            for h in pending:
                h.wait()
            cb = cbufs[ci % 3][0]
            r0, sz = starts[ci], chunk_sizes[ci]

            def row_body(r, tot):
                acc = jnp.zeros((_L,), jnp.float32)
                for s in range(D // _L):
                    d = x_v[r0 + r, pl.ds(s * _L, _L)] - cb[r, pl.ds(s * _L, _L)]
                    acc = acc + d * d
                dist = jnp.sum(acc)
                dist = jnp.minimum(jnp.maximum(dist, 1e-12), 1e12)
                return tot + dist

            tot = lax.fori_loop(0, sz, row_body, tot)
        lane = lax.iota(jnp.int32, _L)
        acc_v[...] = jnp.where(lane == 0, tot * (1.0 / B), 0.0)
        pltpu.sync_copy(acc_v, out_hbm.at[wid])

    return sc_kernel


def kernel(x, labels, centers):
    B, D = x.shape
    sck = _make_sc_kernel(B, D, (128, 128, 128, 128))
    partials = sck(x, labels.astype(jnp.int32), centers)
    return jnp.sum(partials)


# speedup vs baseline: 1.0153x; 1.0153x over previous
import functools

import jax
import jax.numpy as jnp
from jax import lax
from jax.experimental import pallas as pl
from jax.experimental.pallas import tpu as pltpu
from jax.experimental.pallas import tpu_sc as plsc

_NC = 2
_NS = 16
_L = 16
_NW = _NC * _NS


@functools.lru_cache(maxsize=None)
def _make_sc_kernel(B, D, chunk_sizes):
    b_per_w = B // _NW
    assert sum(chunk_sizes) == b_per_w
    starts = []
    off = 0
    for sz in chunk_sizes:
        starts.append(off)
        off += sz
    cmax = max(chunk_sizes)
    n_chunks = len(chunk_sizes)
    mesh = plsc.VectorSubcoreMesh(core_axis_name="c", subcore_axis_name="s")

    @functools.partial(
        pl.kernel,
        mesh=mesh,
        out_type=jax.ShapeDtypeStruct((_NW, _L), jnp.float32),
        scratch_types=[
            pltpu.VMEM((b_per_w,), jnp.int32),
            pltpu.VMEM((b_per_w, D), jnp.float32),
            pltpu.VMEM((cmax, D), jnp.float32),
            pltpu.VMEM((cmax, D), jnp.float32),
            pltpu.VMEM((cmax, D), jnp.float32),
            pltpu.VMEM((_L,), jnp.float32),
            pltpu.SemaphoreType.DMA,
            pltpu.SemaphoreType.DMA,
            pltpu.SemaphoreType.DMA,
        ],
        compiler_params=pltpu.CompilerParams(needs_layout_passes=False),
    )
    def sc_kernel(x_hbm, lab_hbm, cen_hbm, out_hbm,
                  idx_v, x_v, c_v0, c_v1, c_v2, acc_v,
                  sem0, sem1, sem2):
        wid = lax.axis_index("s") * _NC + lax.axis_index("c")
        base = wid * b_per_w
        cbufs = ((c_v0, sem0), (c_v1, sem1), (c_v2, sem2))

        pltpu.sync_copy(lab_hbm.at[pl.ds(base, b_per_w)], idx_v)

        def start(ci):
            cb, sem = cbufs[ci % 3]
            r0, sz = starts[ci], chunk_sizes[ci]
            hc = pltpu.async_copy(
                cen_hbm.at[idx_v.at[pl.ds(r0, sz)]], cb.at[pl.ds(0, sz)], sem)
            hx = pltpu.async_copy(
                x_hbm.at[pl.ds(base + r0, sz)], x_v.at[pl.ds(r0, sz)], sem)
            return hc, hx

        handles = [start(0), start(1)]
        tot = jnp.float32(0.0)
        for ci in range(n_chunks):
            if ci + 2 < n_chunks:
                handles.append(start(ci + 2))
            pending = handles[ci]
            for h in pending:
                h.wait()
            cb = cbufs[ci % 3][0]
            r0, sz = starts[ci], chunk_sizes[ci]

            def row_body(r, tot):
                acc = jnp.zeros((_L,), jnp.float32)
                for s in range(D // _L):
                    d = x_v[r0 + r, pl.ds(s * _L, _L)] - cb[r, pl.ds(s * _L, _L)]
                    acc = acc + d * d
                dist = jnp.sum(acc)
                dist = jnp.minimum(jnp.maximum(dist, 1e-12), 1e12)
                return tot + dist

            tot = lax.fori_loop(0, sz, row_body, tot)
        lane = lax.iota(jnp.int32, _L)
        acc_v[...] = jnp.where(lane == 0, tot * (1.0 / B), 0.0)
        pltpu.sync_copy(acc_v, out_hbm.at[wid])

    return sc_kernel


def kernel(x, labels, centers):
    B, D = x.shape
    sck = _make_sc_kernel(B, D, (32, 64, 128, 144, 144))
    partials = sck(x, labels.astype(jnp.int32), centers)
    return jnp.sum(partials)


# speedup vs baseline: 1.0452x; 1.0295x over previous
import functools

import jax
import jax.numpy as jnp
from jax import lax
from jax.experimental import pallas as pl
from jax.experimental.pallas import tpu as pltpu
from jax.experimental.pallas import tpu_sc as plsc

_NC = 2
_NS = 16
_L = 16
_NW = _NC * _NS


@functools.lru_cache(maxsize=None)
def _make_sc_kernel(B, D, chunk_sizes):
    b_per_w = B // _NW
    assert sum(chunk_sizes) == b_per_w
    starts = []
    off = 0
    for sz in chunk_sizes:
        starts.append(off)
        off += sz
    cmax = max(chunk_sizes)
    n_chunks = len(chunk_sizes)
    mesh = plsc.VectorSubcoreMesh(core_axis_name="c", subcore_axis_name="s")

    @functools.partial(
        pl.kernel,
        mesh=mesh,
        out_type=jax.ShapeDtypeStruct((_NW, _L), jnp.float32),
        scratch_types=[
            pltpu.VMEM((b_per_w,), jnp.int32),
            pltpu.VMEM((b_per_w, D), jnp.float32),
            pltpu.VMEM((cmax, D), jnp.float32),
            pltpu.VMEM((cmax, D), jnp.float32),
            pltpu.VMEM((cmax, D), jnp.float32),
            pltpu.VMEM((_L,), jnp.float32),
            pltpu.SemaphoreType.DMA,
            pltpu.SemaphoreType.DMA,
            pltpu.SemaphoreType.DMA,
        ],
        compiler_params=pltpu.CompilerParams(needs_layout_passes=False),
    )
    def sc_kernel(x_hbm, lab_hbm, cen_hbm, out_hbm,
                  idx_v, x_v, c_v0, c_v1, c_v2, acc_v,
                  sem0, sem1, sem2):
        wid = lax.axis_index("s") * _NC + lax.axis_index("c")
        base = wid * b_per_w
        cbufs = ((c_v0, sem0), (c_v1, sem1), (c_v2, sem2))

        pltpu.sync_copy(lab_hbm.at[pl.ds(base, b_per_w)], idx_v)

        def start(ci):
            cb, sem = cbufs[ci % 3]
            r0, sz = starts[ci], chunk_sizes[ci]
            hc = pltpu.async_copy(
                cen_hbm.at[idx_v.at[pl.ds(r0, sz)]], cb.at[pl.ds(0, sz)], sem)
            hx = pltpu.async_copy(
                x_hbm.at[pl.ds(base + r0, sz)], x_v.at[pl.ds(r0, sz)], sem)
            return hc, hx

        handles = [start(0), start(1)]
        tot = jnp.float32(0.0)
        for ci in range(n_chunks):
            if ci + 2 < n_chunks:
                handles.append(start(ci + 2))
            pending = handles[ci]
            for h in pending:
                h.wait()
            cb = cbufs[ci % 3][0]
            r0, sz = starts[ci], chunk_sizes[ci]

            def row_body(r, tot):
                acc = jnp.zeros((_L,), jnp.float32)
                for s in range(D // _L):
                    d = x_v[r0 + r, pl.ds(s * _L, _L)] - cb[r, pl.ds(s * _L, _L)]
                    acc = acc + d * d
                dist = jnp.sum(acc)
                dist = jnp.minimum(jnp.maximum(dist, 1e-12), 1e12)
                return tot + dist

            tot = lax.fori_loop(0, sz, row_body, tot)
        lane = lax.iota(jnp.int32, _L)
        acc_v[...] = jnp.where(lane == 0, tot * (1.0 / B), 0.0)
        pltpu.sync_copy(acc_v, out_hbm.at[wid])

    return sc_kernel


def kernel(x, labels, centers):
    B, D = x.shape
    sck = _make_sc_kernel(B, D, (32, 96, 128, 128, 128))
    partials = sck(x, labels.astype(jnp.int32), centers)
    return jnp.sum(partials)
